# baseline (speedup 1.0000x reference)
import functools

import jax
import jax.numpy as jnp
from jax.experimental import pallas as pl
from jax.experimental.pallas import tpu as pltpu

N_PAIRS = 3_200_000
TC_BLK = 25_600
GRID = N_PAIRS // TC_BLK


def _tree_lookup(code, bits, t):
    lvl = list(t)
    for b in bits:
        lvl = [jnp.where(b, lvl[2 * k + 1], lvl[2 * k])
               for k in range(len(lvl) // 2)]
    return lvl[0]


def _tc_body(tbl_inv_ref, tbl_cf_ref, dr_ref, zi_ref, zj_ref, out_ref):
    code = zi_ref[...] * 4 + zj_ref[...]
    bits = [(code & (1 << k)) != 0 for k in range(4)]
    inv_sig = _tree_lookup(code, bits, [tbl_inv_ref[k] for k in range(16)])
    cf = _tree_lookup(code, bits, [tbl_cf_ref[k] for k in range(16)])
    x = jnp.maximum(1.0 - dr_ref[...] * inv_sig, 0.0)
    out_ref[...] = cf * x * x


@functools.cache
def _tc_call():
    return pl.pallas_call(
        _tc_body,
        grid=(GRID,),
        in_specs=[
            pl.BlockSpec(memory_space=pltpu.SMEM),
            pl.BlockSpec(memory_space=pltpu.SMEM),
            pl.BlockSpec((TC_BLK,), lambda i: (i,)),
            pl.BlockSpec((TC_BLK,), lambda i: (i,)),
            pl.BlockSpec((TC_BLK,), lambda i: (i,)),
        ],
        out_specs=pl.BlockSpec((TC_BLK,), lambda i: (i,)),
        out_shape=jax.ShapeDtypeStruct((N_PAIRS,), jnp.float32),
    )


def kernel(dr, zi, zj, z_to_idx, sigma_matrix, epsilon_matrix, alpha_matrix):
    sig = sigma_matrix[z_to_idx[:, None], z_to_idx[None, :]]
    eps = epsilon_matrix[z_to_idx[:, None], z_to_idx[None, :]]
    alp = alpha_matrix[z_to_idx[:, None], z_to_idx[None, :]]
    inv_sigma_t = (1.0 / sig).reshape(-1)
    coeff_t = (eps / alp).reshape(-1)
    return _tc_call()(inv_sigma_t, coeff_t, dr, zi, zj)


# speedup vs baseline: 2.2451x; 2.2451x over previous
import functools

import jax
import jax.numpy as jnp
from jax.experimental import pallas as pl
from jax.experimental.pallas import tpu as pltpu

N_PAIRS = 3_200_000
ROWS = N_PAIRS // 128
BLK_R = 1_000
GRID = ROWS // BLK_R


def _tree_lookup(bits, t):
    lvl = list(t)
    for b in bits:
        lvl = [jnp.where(b, lvl[2 * k + 1], lvl[2 * k])
               for k in range(len(lvl) // 2)]
    return lvl[0]


def _tc_body(tbl_inv_ref, tbl_cf_ref, dr_ref, zi_ref, zj_ref, out_ref):
    code = zi_ref[...] * 4 + zj_ref[...]
    bits = [(code & (1 << k)) != 0 for k in range(4)]
    inv_sig = _tree_lookup(bits, [tbl_inv_ref[k] for k in range(16)])
    cf = _tree_lookup(bits, [tbl_cf_ref[k] for k in range(16)])
    x = jnp.maximum(1.0 - dr_ref[...] * inv_sig, 0.0)
    out_ref[...] = cf * x * x


@functools.cache
def _tc_call():
    return pl.pallas_call(
        _tc_body,
        grid=(GRID,),
        in_specs=[
            pl.BlockSpec(memory_space=pltpu.SMEM),
            pl.BlockSpec(memory_space=pltpu.SMEM),
            pl.BlockSpec((BLK_R, 128), lambda i: (i, 0)),
            pl.BlockSpec((BLK_R, 128), lambda i: (i, 0)),
            pl.BlockSpec((BLK_R, 128), lambda i: (i, 0)),
        ],
        out_specs=pl.BlockSpec((BLK_R, 128), lambda i: (i, 0)),
        out_shape=jax.ShapeDtypeStruct((ROWS, 128), jnp.float32),
    )


def kernel(dr, zi, zj, z_to_idx, sigma_matrix, epsilon_matrix, alpha_matrix):
    sig = sigma_matrix[z_to_idx[:, None], z_to_idx[None, :]]
    eps = epsilon_matrix[z_to_idx[:, None], z_to_idx[None, :]]
    alp = alpha_matrix[z_to_idx[:, None], z_to_idx[None, :]]
    inv_sigma_t = (1.0 / sig).reshape(-1)
    coeff_t = (eps / alp).reshape(-1)
    out2d = _tc_call()(inv_sigma_t, coeff_t,
                       dr.reshape(ROWS, 128),
                       zi.reshape(ROWS, 128),
                       zj.reshape(ROWS, 128))
    return out2d.reshape(N_PAIRS)
